# Initial kernel scaffold; baseline (speedup 1.0000x reference)
import jax
import jax.numpy as jnp
from jax.experimental import pallas as pl


def kernel(student, teacher):
    raise NotImplementedError("write your pallas kernel here")


# speedup vs baseline: 22.1153x; 22.1153x over previous
import functools

import jax
import jax.numpy as jnp
from jax.experimental import pallas as pl

K = 256
B = 64
N = 32768
MIN32 = -2147483648


def _ikey(x):
    u = jax.lax.bitcast_convert_type(x, jnp.int32)
    return u ^ ((u >> 31) & jnp.int32(0x7FFFFFFF))


def _colsum_body(s_ref, t_ref, out_ref):
    d = s_ref[...] - t_ref[...]
    out_ref[...] = jnp.sum(d * d, axis=0, keepdims=True)


def _select_body(s_ref, t_out, c_out):
    ikey = _ikey(s_ref[...])

    def bit_step(i, tb):
        bit = jnp.left_shift(jnp.int32(1), 31 - i)
        cb = tb | bit
        thr = cb ^ jnp.int32(MIN32)
        cnt = jnp.sum((ikey < thr).astype(jnp.int32), axis=1, keepdims=True)
        return jnp.where(cnt >= K, tb, cb)

    tb = jax.lax.fori_loop(0, 32, bit_step, jnp.zeros((B, 1), jnp.int32))
    t = tb ^ jnp.int32(MIN32)
    cnt_lt = jnp.sum((ikey < t).astype(jnp.int32), axis=1, keepdims=True)
    n_t = K - cnt_lt
    col = jax.lax.broadcasted_iota(jnp.int32, (B, N), 1)
    tie_cols = jnp.where(ikey == t, col, jnp.int32(65535))

    def tie_step(i, cb_acc):
        bit = jnp.left_shift(jnp.int32(1), 15 - i)
        cb = cb_acc | bit
        cnt = jnp.sum((tie_cols < cb).astype(jnp.int32), axis=1, keepdims=True)
        return jnp.where(cnt >= n_t, cb_acc, cb)

    cutoff = jax.lax.fori_loop(0, 16, tie_step, jnp.zeros((B, 1), jnp.int32))
    t_out[...] = t
    c_out[...] = cutoff


def _combine_body(s_ref, cs_ref, t_ref, c_ref, out_ref):
    pid = pl.program_id(0)
    blk = s_ref.shape[1]
    ikey = _ikey(s_ref[...])
    t = t_ref[...]
    cutoff = c_ref[...]
    col = jax.lax.broadcasted_iota(jnp.int32, (B, blk), 1) + pid * blk
    sel = (ikey < t) | ((ikey == t) & (col <= cutoff))
    part = jnp.sum(jnp.where(sel, cs_ref[...], 0.0)).reshape(1, 1)

    @pl.when(pid == 0)
    def _():
        out_ref[...] = jnp.zeros((1, 1), jnp.float32)

    out_ref[...] += part


def kernel(student, teacher):
    colsum = pl.pallas_call(
        _colsum_body,
        grid=(8,),
        in_specs=[
            pl.BlockSpec((B, N // 8), lambda i: (0, i)),
            pl.BlockSpec((B, N // 8), lambda i: (0, i)),
        ],
        out_specs=pl.BlockSpec((1, N // 8), lambda i: (0, i)),
        out_shape=jax.ShapeDtypeStruct((1, N), jnp.float32),
    )(student, teacher)

    t, cutoff = pl.pallas_call(
        _select_body,
        out_shape=(
            jax.ShapeDtypeStruct((B, 1), jnp.int32),
            jax.ShapeDtypeStruct((B, 1), jnp.int32),
        ),
    )(student)

    out = pl.pallas_call(
        _combine_body,
        grid=(8,),
        in_specs=[
            pl.BlockSpec((B, N // 8), lambda i: (0, i)),
            pl.BlockSpec((1, N // 8), lambda i: (0, i)),
            pl.BlockSpec((B, 1), lambda i: (0, 0)),
            pl.BlockSpec((B, 1), lambda i: (0, 0)),
        ],
        out_specs=pl.BlockSpec((1, 1), lambda i: (0, 0)),
        out_shape=jax.ShapeDtypeStruct((1, 1), jnp.float32),
    )(student, colsum, t, cutoff)
    return out[0, 0]
